# baseline (speedup 1.0000x reference)
import functools

import jax
import jax.numpy as jnp
from jax import lax
from jax.experimental import pallas as pl
from jax.experimental.pallas import tpu as pltpu
from jax.experimental.pallas import tpu_sc as plsc

N = 32768
K = 16
HID = 32
DF = 64
T = 3


QB = 256
CB = 2048
NCHUNK = N // CB
_BIGI = N
_INF = float("inf")


def _extract_topk(vals, gidx, nk):
    vs, ix = [], []
    for _ in range(nk):
        m = jnp.min(vals, axis=1, keepdims=True)
        tie = vals <= m
        gi = jnp.min(jnp.where(tie, gidx, _BIGI), axis=1, keepdims=True)
        sel = tie & (gidx == gi)
        vals = jnp.where(sel, _INF, vals)
        vs.append(m)
        ix.append(gi)
    return jnp.concatenate(vs, axis=1), jnp.concatenate(ix, axis=1)


def _knn_body(cq_ref, ct_ref, idx_ref):
    cq = cq_ref[...]
    sqq = jnp.sum(cq * cq, axis=1, keepdims=True)
    iota_c = lax.broadcasted_iota(jnp.int32, (QB, CB), 1)

    def chunk_body(c, carry):
        runv, runi = carry
        ct = ct_ref[:, pl.ds(c * CB, CB)]
        sqc = jnp.sum(ct * ct, axis=0, keepdims=True)
        d = sqq + sqc - 2.0 * jnp.dot(cq, ct, preferred_element_type=jnp.float32)
        gidx = iota_c + c * CB
        tau = runv[:, K - 1:K]
        anyq = jnp.max(jnp.where(d < tau, 1.0, 0.0)) > 0.5

        def do(_):
            cv, ci = _extract_topk(d, gidx, K)
            mv = jnp.concatenate([runv, cv], axis=1)
            mi = jnp.concatenate([runi, ci], axis=1)
            return _extract_topk(mv, mi, K)

        def skip(_):
            return runv, runi

        return lax.cond(anyq, do, skip, 0)

    home = pl.program_id(0) // (CB // QB)
    offsets = [0]
    for o in range(1, NCHUNK // 2 + 1):
        offsets.append(o)
        if o != NCHUNK // 2:
            offsets.append(-o)
    carry = (jnp.full((QB, K), _INF, jnp.float32),
             jnp.full((QB, K), _BIGI, jnp.int32))
    for off in offsets:
        c = lax.rem(home + jnp.int32(off + NCHUNK), jnp.int32(NCHUNK))
        carry = chunk_body(c, carry)
    idx_ref[...] = carry[1]


def _knn(coords_pad, coords_t):
    return pl.pallas_call(
        _knn_body,
        grid=(N // QB,),
        in_specs=[
            pl.BlockSpec((QB, 8), lambda i: (i, 0)),
            pl.BlockSpec((8, N), lambda i: (0, 0)),
        ],
        out_specs=pl.BlockSpec((QB, K), lambda i: (i, 0)),
        out_shape=jax.ShapeDtypeStruct((N, K), jnp.int32),
    )(coords_pad, coords_t)


_E = N * K
_CH = 128
_G = 4
_NW = 32


def _make_gather(D, nrows):
    nrowblk = nrows // _CH
    ngroups = nrowblk // _G
    gpw = ngroups // _NW
    mesh = plsc.VectorSubcoreMesh(core_axis_name="c", subcore_axis_name="s")

    @functools.partial(
        pl.kernel,
        out_type=jax.ShapeDtypeStruct((nrowblk, _CH, D), jnp.float32),
        mesh=mesh,
        compiler_params=pltpu.CompilerParams(use_tc_tiling_on_sc=False),
        scratch_types=[
            pltpu.VMEM((_G, _CH), jnp.int32),
            pltpu.VMEM((_G, _CH, D), jnp.float32),
            [pltpu.SemaphoreType.DMA] * _G,
        ],
    )
    def gather_kernel(table_hbm, idx_hbm, out_hbm, idx_v, rows_v, sems):
        wid = lax.axis_index("s") * 2 + lax.axis_index("c")

        def group(g, carry):
            gbase = wid * gpw + g
            for b in range(_G):
                pltpu.sync_copy(idx_hbm.at[gbase * _G + b], idx_v.at[b])
            copies = []
            for b in range(_G):
                copies.append(
                    pltpu.async_copy(table_hbm.at[idx_v.at[b]], rows_v.at[b],
                                     sems[b]))
            for b in range(_G):
                copies[b].wait()
            pltpu.sync_copy(rows_v, out_hbm.at[pl.ds(gbase * _G, _G)])
            return carry

        lax.fori_loop(0, gpw, group, 0)

    def run(table, idx_flat):
        idx2 = idx_flat.reshape(nrowblk, _CH)
        out = gather_kernel(table, idx2)
        return out.reshape(nrows, D)

    return run


_gather_cache = {}


def _gather_rows(table, idx_flat):
    key = (table.shape[1], idx_flat.shape[0])
    if key not in _gather_cache:
        _gather_cache[key] = _make_gather(*key)
    return _gather_cache[key](table, idx_flat)


Q2 = 512
EB = Q2 * K


def _gelu(x):
    return 0.5 * x * (1.0 + lax.erf(x * 0.7071067811865476))


def _sigmoid(x):
    return 1.0 / (1.0 + jnp.exp(-x))


def _green_g(rel8, vi, vj, w_r, w_i, w_j, b1, w2, b2, w3r, b3):
    h = rel8 @ w_r + vi @ w_i + vj @ w_j + b1
    h = _gelu(h)
    h = _gelu(h @ w2 + b2)
    return _sigmoid(jnp.sum(h * w3r, axis=1, keepdims=True) + b3)


TW = 48


def _round_body(cq_ref, vq_ref, cj_ref, vj_ref,
                w_r_ref, w_i_ref, w_j_ref, b1_ref, w2_ref, b2_ref,
                w3r_ref, b3_ref, ww_ref, s_ref, b_ref, o_ref):
    cq8 = cq_ref[:, 0:8]
    vq = vq_ref[...]
    cj8 = cj_ref[:, 0:8]
    vj = vj_ref[...]

    rel8 = cj8 - jnp.broadcast_to(cq8[:, None, :], (Q2, K, 8)).reshape(EB, 8)
    vi = jnp.broadcast_to(vq[:, None, :], (Q2, K, HID)).reshape(EB, HID)
    g = _green_g(rel8, vi, vj, w_r_ref[...], w_i_ref[...], w_j_ref[...],
                 b1_ref[...], w2_ref[...], b2_ref[...], w3r_ref[...],
                 b3_ref[...])
    integral = jnp.sum((g * vj).reshape(Q2, K, HID), axis=1) * (1.0 / K)
    pre = jnp.maximum(
        integral + jnp.dot(vq, ww_ref[...], preferred_element_type=jnp.float32),
        0.0)
    mu = jnp.mean(pre, axis=1, keepdims=True)
    var = jnp.mean(pre * pre, axis=1, keepdims=True) - mu * mu
    o_ref[...] = (pre - mu) * lax.rsqrt(var + 1e-5) * s_ref[...] + b_ref[...]


def _final_body(cq_ref, vq_ref, cj_ref, vj_ref,
                w_r_ref, w_i_ref, w_j_ref, b1_ref, w2_ref, b2_ref,
                w3r_ref, b3_ref, p1_ref, p1b_ref, p2r_ref, p2b_ref,
                s_ref, w_ref):
    cq8 = cq_ref[:, 0:8]
    vq = vq_ref[...]
    cj8 = cj_ref[:, 0:8]
    vj = vj_ref[...]
    rel8 = cj8 - jnp.broadcast_to(cq8[:, None, :], (Q2, K, 8)).reshape(EB, 8)
    vi = jnp.broadcast_to(vq[:, None, :], (Q2, K, HID)).reshape(EB, HID)
    g = _green_g(rel8, vi, vj, w_r_ref[...], w_i_ref[...], w_j_ref[...],
                 b1_ref[...], w2_ref[...], b2_ref[...], w3r_ref[...],
                 b3_ref[...])
    w_ref[...] = g.reshape(Q2, K)
    h = _gelu(jnp.dot(vq, p1_ref[...], preferred_element_type=jnp.float32)
              + p1b_ref[...])
    s_ref[...] = _sigmoid(jnp.sum(h * p2r_ref[...], axis=1, keepdims=True)
                          + p2b_ref[...])


def _lift_body(cq_ref, f_ref, wc_ref, wf_ref, b_ref, o_ref):
    o_ref[...] = (jnp.dot(cq_ref[:, 0:8], wc_ref[...],
                          preferred_element_type=jnp.float32)
                  + jnp.dot(f_ref[...], wf_ref[...],
                            preferred_element_type=jnp.float32)
                  + b_ref[...])


def _wspec(shape):
    return pl.BlockSpec(shape, lambda i: tuple(0 for _ in shape))


def _morton_perm(coords):
    q = jnp.clip((coords * 1024.0).astype(jnp.int32), 0, 1023)

    def spread(x):
        x = (x | (x << 16)) & 0x030000FF
        x = (x | (x << 8)) & 0x0300F00F
        x = (x | (x << 4)) & 0x030C30C3
        x = (x | (x << 2)) & 0x09249249
        return x

    key = spread(q[:, 0]) | (spread(q[:, 1]) << 1) | (spread(q[:, 2]) << 2)
    perm = jnp.argsort(key).astype(jnp.int32)
    inv = jnp.zeros((N,), jnp.int32).at[perm].set(
        jnp.arange(N, dtype=jnp.int32))
    ident = jnp.arange(N, dtype=jnp.int32)
    return ident, ident


def kernel(coords, feat, lift_w, lift_b, gk1_w, gk1_b, gk2_w, gk2_b, gk3_w,
           gk3_b, W_w, ln_scale, ln_bias, proj1_w, proj1_b, proj2_w, proj2_b):
    perm, inv = _morton_perm(coords)
    coords_pad16 = jnp.pad(coords, ((0, 0), (0, 13)))
    table0 = jnp.concatenate([coords_pad16, feat], axis=1)
    g0 = _gather_rows(table0, perm)

    coords_s8 = g0[:, 0:8]
    idx_s = _knn(coords_s8, coords_s8.T)
    idx_flat = idx_s.reshape(_E)

    coords_s16 = g0[:, 0:16]
    feat_s = g0[:, 16:80]
    v = pl.pallas_call(
        _lift_body,
        grid=(N // Q2,),
        in_specs=[
            pl.BlockSpec((Q2, 16), lambda i: (i, 0)),
            pl.BlockSpec((Q2, DF), lambda i: (i, 0)),
            _wspec((8, HID)),
            _wspec((DF, HID)),
            _wspec((1, HID)),
        ],
        out_specs=pl.BlockSpec((Q2, HID), lambda i: (i, 0)),
        out_shape=jax.ShapeDtypeStruct((N, HID), jnp.float32),
    )(coords_s16, feat_s, jnp.pad(lift_w[0:3], ((0, 5), (0, 0))), lift_w[3:],
      lift_b[None, :])

    w_r = jnp.pad(gk1_w[0:3], ((0, 5), (0, 0)))
    w_i = gk1_w[3:3 + HID]
    w_j = gk1_w[3 + HID:]
    b1 = gk1_b[None, :]
    b2 = gk2_b[None, :]
    w3r = gk3_w.T
    b3 = gk3_b[None, :]

    green_specs = [
        _wspec((8, HID)), _wspec((HID, HID)), _wspec((HID, HID)),
        _wspec((1, HID)), _wspec((HID, HID)), _wspec((1, HID)),
        _wspec((1, HID)), _wspec((1, 1)),
    ]

    perm_rep_f = jnp.broadcast_to(perm[:, None].astype(jnp.float32), (N, 16))

    row_specs = [
        pl.BlockSpec((Q2, 16), lambda i: (i, 0)),
        pl.BlockSpec((Q2, HID), lambda i: (i, 0)),
        pl.BlockSpec((EB, 16), lambda i: (i, 0)),
        pl.BlockSpec((EB, HID), lambda i: (i, 0)),
    ]

    cj = _gather_rows(coords_s16, idx_flat)
    idxvals_f = _gather_rows(perm_rep_f, idx_flat)[:, 0].reshape(N, K)
    for t in range(T):
        vj = _gather_rows(v, idx_flat)
        v = pl.pallas_call(
            _round_body,
            grid=(N // Q2,),
            in_specs=row_specs + green_specs + [
                _wspec((HID, HID)), _wspec((1, HID)), _wspec((1, HID)),
            ],
            out_specs=pl.BlockSpec((Q2, HID), lambda i: (i, 0)),
            out_shape=jax.ShapeDtypeStruct((N, HID), jnp.float32),
        )(coords_s16, v, cj, vj, w_r, w_i, w_j, b1,
          gk2_w, b2, w3r, b3, W_w, ln_scale[t][None, :], ln_bias[t][None, :])

    vjf = _gather_rows(v, idx_flat)
    scores_s, w_ij_s = pl.pallas_call(
        _final_body,
        grid=(N // Q2,),
        in_specs=row_specs + green_specs + [
            _wspec((HID, HID // 2)), _wspec((1, HID // 2)),
            _wspec((1, HID // 2)), _wspec((1, 1)),
        ],
        out_specs=[
            pl.BlockSpec((Q2, 1), lambda i: (i, 0)),
            pl.BlockSpec((Q2, K), lambda i: (i, 0)),
        ],
        out_shape=[
            jax.ShapeDtypeStruct((N, 1), jnp.float32),
            jax.ShapeDtypeStruct((N, K), jnp.float32),
        ],
    )(coords_s16, v, cj, vjf, w_r, w_i, w_j, b1, gk2_w, b2, w3r, b3,
      proj1_w, proj1_b[None, :], proj2_w.T, proj2_b[None, :])

    big = jnp.concatenate(
        [v, w_ij_s, idxvals_f,
         jnp.broadcast_to(scores_s, (N, 16))], axis=1)
    og = _gather_rows(big, inv)
    v = og[:, 0:HID]
    w_ij = og[:, HID:HID + K]
    idx = og[:, 48:64].astype(jnp.int32)
    scores = og[:, 64:65]
    return (scores, idx, w_ij, v)


# speedup vs baseline: 1.1715x; 1.1715x over previous
import functools

import jax
import jax.numpy as jnp
from jax import lax
from jax.experimental import pallas as pl
from jax.experimental.pallas import tpu as pltpu
from jax.experimental.pallas import tpu_sc as plsc

N = 32768
K = 16
HID = 32
DF = 64
T = 3


QB = 256
CB = 2048
NCHUNK = N // CB
_BIGI = N
_INF = float("inf")


def _extract_topk(vals, gidx, nk):
    vs, ix = [], []
    for _ in range(nk):
        m = jnp.min(vals, axis=1, keepdims=True)
        tie = vals <= m
        gi = jnp.min(jnp.where(tie, gidx, _BIGI), axis=1, keepdims=True)
        sel = tie & (gidx == gi)
        vals = jnp.where(sel, _INF, vals)
        vs.append(m)
        ix.append(gi)
    return jnp.concatenate(vs, axis=1), jnp.concatenate(ix, axis=1)


def _knn_body(cq_ref, ct_ref, idx_ref):
    cq = cq_ref[...]
    sqq = jnp.sum(cq * cq, axis=1, keepdims=True)
    iota_c = lax.broadcasted_iota(jnp.int32, (QB, CB), 1)

    def chunk_body(c, carry):
        runv, runi = carry
        ct = ct_ref[:, pl.ds(c * CB, CB)]
        sqc = jnp.sum(ct * ct, axis=0, keepdims=True)
        d = sqq + sqc - 2.0 * jnp.dot(cq, ct, preferred_element_type=jnp.float32)
        gidx = iota_c + c * CB
        cv, ci = _extract_topk(d, gidx, K)
        mv = jnp.concatenate([runv, cv], axis=1)
        mi = jnp.concatenate([runi, ci], axis=1)
        return _extract_topk(mv, mi, K)

    home = pl.program_id(0) // (CB // QB)
    offsets = [0]
    for o in range(1, NCHUNK // 2 + 1):
        offsets.append(o)
        if o != NCHUNK // 2:
            offsets.append(-o)
    carry = (jnp.full((QB, K), _INF, jnp.float32),
             jnp.full((QB, K), _BIGI, jnp.int32))
    for off in offsets:
        c = lax.rem(home + jnp.int32(off + NCHUNK), jnp.int32(NCHUNK))
        carry = chunk_body(c, carry)
    idx_ref[...] = carry[1]


def _knn(coords_pad, coords_t):
    return pl.pallas_call(
        _knn_body,
        grid=(N // QB,),
        in_specs=[
            pl.BlockSpec((QB, 8), lambda i: (i, 0)),
            pl.BlockSpec((8, N), lambda i: (0, 0)),
        ],
        out_specs=pl.BlockSpec((QB, K), lambda i: (i, 0)),
        out_shape=jax.ShapeDtypeStruct((N, K), jnp.int32),
    )(coords_pad, coords_t)


_E = N * K
_CH = 128
_G = 4
_NW = 32


def _make_gather(D, nrows):
    nrowblk = nrows // _CH
    ngroups = nrowblk // _G
    gpw = ngroups // _NW
    mesh = plsc.VectorSubcoreMesh(core_axis_name="c", subcore_axis_name="s")

    @functools.partial(
        pl.kernel,
        out_type=jax.ShapeDtypeStruct((nrowblk, _CH, D), jnp.float32),
        mesh=mesh,
        compiler_params=pltpu.CompilerParams(use_tc_tiling_on_sc=False),
        scratch_types=[
            pltpu.VMEM((_G, _CH), jnp.int32),
            pltpu.VMEM((_G, _CH, D), jnp.float32),
            [pltpu.SemaphoreType.DMA] * _G,
        ],
    )
    def gather_kernel(table_hbm, idx_hbm, out_hbm, idx_v, rows_v, sems):
        wid = lax.axis_index("s") * 2 + lax.axis_index("c")

        def group(g, carry):
            gbase = wid * gpw + g
            for b in range(_G):
                pltpu.sync_copy(idx_hbm.at[gbase * _G + b], idx_v.at[b])
            copies = []
            for b in range(_G):
                copies.append(
                    pltpu.async_copy(table_hbm.at[idx_v.at[b]], rows_v.at[b],
                                     sems[b]))
            for b in range(_G):
                copies[b].wait()
            pltpu.sync_copy(rows_v, out_hbm.at[pl.ds(gbase * _G, _G)])
            return carry

        lax.fori_loop(0, gpw, group, 0)

    def run(table, idx_flat):
        idx2 = idx_flat.reshape(nrowblk, _CH)
        out = gather_kernel(table, idx2)
        return out.reshape(nrows, D)

    return run


_gather_cache = {}


def _gather_rows(table, idx_flat):
    key = (table.shape[1], idx_flat.shape[0])
    if key not in _gather_cache:
        _gather_cache[key] = _make_gather(*key)
    return _gather_cache[key](table, idx_flat)


Q2 = 512
EB = Q2 * K


def _gelu(x):
    return 0.5 * x * (1.0 + lax.erf(x * 0.7071067811865476))


def _sigmoid(x):
    return 1.0 / (1.0 + jnp.exp(-x))


def _green_g(rel8, vi, vj, w_r, w_i, w_j, b1, w2, b2, w3r, b3):
    h = rel8 @ w_r + vi @ w_i + vj @ w_j + b1
    h = _gelu(h)
    h = _gelu(h @ w2 + b2)
    return _sigmoid(jnp.sum(h * w3r, axis=1, keepdims=True) + b3)


TW = 48


def _round_body(cq_ref, vq_ref, cj_ref, vj_ref,
                w_r_ref, w_i_ref, w_j_ref, b1_ref, w2_ref, b2_ref,
                w3r_ref, b3_ref, ww_ref, s_ref, b_ref, o_ref):
    cq8 = cq_ref[:, 0:8]
    vq = vq_ref[...]
    cj8 = cj_ref[:, 0:8]
    vj = vj_ref[...]

    rel8 = cj8 - jnp.broadcast_to(cq8[:, None, :], (Q2, K, 8)).reshape(EB, 8)
    vi = jnp.broadcast_to(vq[:, None, :], (Q2, K, HID)).reshape(EB, HID)
    g = _green_g(rel8, vi, vj, w_r_ref[...], w_i_ref[...], w_j_ref[...],
                 b1_ref[...], w2_ref[...], b2_ref[...], w3r_ref[...],
                 b3_ref[...])
    integral = jnp.sum((g * vj).reshape(Q2, K, HID), axis=1) * (1.0 / K)
    pre = jnp.maximum(
        integral + jnp.dot(vq, ww_ref[...], preferred_element_type=jnp.float32),
        0.0)
    mu = jnp.mean(pre, axis=1, keepdims=True)
    var = jnp.mean(pre * pre, axis=1, keepdims=True) - mu * mu
    o_ref[...] = (pre - mu) * lax.rsqrt(var + 1e-5) * s_ref[...] + b_ref[...]


def _final_body(cq_ref, vq_ref, cj_ref, vj_ref,
                w_r_ref, w_i_ref, w_j_ref, b1_ref, w2_ref, b2_ref,
                w3r_ref, b3_ref, p1_ref, p1b_ref, p2r_ref, p2b_ref,
                s_ref, w_ref):
    cq8 = cq_ref[:, 0:8]
    vq = vq_ref[...]
    cj8 = cj_ref[:, 0:8]
    vj = vj_ref[...]
    rel8 = cj8 - jnp.broadcast_to(cq8[:, None, :], (Q2, K, 8)).reshape(EB, 8)
    vi = jnp.broadcast_to(vq[:, None, :], (Q2, K, HID)).reshape(EB, HID)
    g = _green_g(rel8, vi, vj, w_r_ref[...], w_i_ref[...], w_j_ref[...],
                 b1_ref[...], w2_ref[...], b2_ref[...], w3r_ref[...],
                 b3_ref[...])
    w_ref[...] = g.reshape(Q2, K)
    h = _gelu(jnp.dot(vq, p1_ref[...], preferred_element_type=jnp.float32)
              + p1b_ref[...])
    s_ref[...] = _sigmoid(jnp.sum(h * p2r_ref[...], axis=1, keepdims=True)
                          + p2b_ref[...])


def _lift_body(cq_ref, f_ref, wc_ref, wf_ref, b_ref, o_ref):
    o_ref[...] = (jnp.dot(cq_ref[:, 0:8], wc_ref[...],
                          preferred_element_type=jnp.float32)
                  + jnp.dot(f_ref[...], wf_ref[...],
                            preferred_element_type=jnp.float32)
                  + b_ref[...])


def _wspec(shape):
    return pl.BlockSpec(shape, lambda i: tuple(0 for _ in shape))


def _morton_perm(coords):
    q = jnp.clip((coords * 1024.0).astype(jnp.int32), 0, 1023)

    def spread(x):
        x = (x | (x << 16)) & 0x030000FF
        x = (x | (x << 8)) & 0x0300F00F
        x = (x | (x << 4)) & 0x030C30C3
        x = (x | (x << 2)) & 0x09249249
        return x

    key = spread(q[:, 0]) | (spread(q[:, 1]) << 1) | (spread(q[:, 2]) << 2)
    perm = jnp.argsort(key).astype(jnp.int32)
    inv = jnp.zeros((N,), jnp.int32).at[perm].set(
        jnp.arange(N, dtype=jnp.int32))
    ident = jnp.arange(N, dtype=jnp.int32)
    return ident, ident


def kernel(coords, feat, lift_w, lift_b, gk1_w, gk1_b, gk2_w, gk2_b, gk3_w,
           gk3_b, W_w, ln_scale, ln_bias, proj1_w, proj1_b, proj2_w, proj2_b):
    perm, inv = _morton_perm(coords)
    coords_pad16 = jnp.pad(coords, ((0, 0), (0, 13)))
    table0 = jnp.concatenate([coords_pad16, feat], axis=1)
    g0 = _gather_rows(table0, perm)

    coords_s8 = g0[:, 0:8]
    idx_s = _knn(coords_s8, coords_s8.T)
    idx_flat = idx_s.reshape(_E)

    coords_s16 = g0[:, 0:16]
    feat_s = g0[:, 16:80]
    v = pl.pallas_call(
        _lift_body,
        grid=(N // Q2,),
        in_specs=[
            pl.BlockSpec((Q2, 16), lambda i: (i, 0)),
            pl.BlockSpec((Q2, DF), lambda i: (i, 0)),
            _wspec((8, HID)),
            _wspec((DF, HID)),
            _wspec((1, HID)),
        ],
        out_specs=pl.BlockSpec((Q2, HID), lambda i: (i, 0)),
        out_shape=jax.ShapeDtypeStruct((N, HID), jnp.float32),
    )(coords_s16, feat_s, jnp.pad(lift_w[0:3], ((0, 5), (0, 0))), lift_w[3:],
      lift_b[None, :])

    w_r = jnp.pad(gk1_w[0:3], ((0, 5), (0, 0)))
    w_i = gk1_w[3:3 + HID]
    w_j = gk1_w[3 + HID:]
    b1 = gk1_b[None, :]
    b2 = gk2_b[None, :]
    w3r = gk3_w.T
    b3 = gk3_b[None, :]

    green_specs = [
        _wspec((8, HID)), _wspec((HID, HID)), _wspec((HID, HID)),
        _wspec((1, HID)), _wspec((HID, HID)), _wspec((1, HID)),
        _wspec((1, HID)), _wspec((1, 1)),
    ]

    perm_rep_f = jnp.broadcast_to(perm[:, None].astype(jnp.float32), (N, 16))

    row_specs = [
        pl.BlockSpec((Q2, 16), lambda i: (i, 0)),
        pl.BlockSpec((Q2, HID), lambda i: (i, 0)),
        pl.BlockSpec((EB, 16), lambda i: (i, 0)),
        pl.BlockSpec((EB, HID), lambda i: (i, 0)),
    ]

    cj = _gather_rows(coords_s16, idx_flat)
    idxvals_f = _gather_rows(perm_rep_f, idx_flat)[:, 0].reshape(N, K)
    for t in range(T):
        vj = _gather_rows(v, idx_flat)
        v = pl.pallas_call(
            _round_body,
            grid=(N // Q2,),
            in_specs=row_specs + green_specs + [
                _wspec((HID, HID)), _wspec((1, HID)), _wspec((1, HID)),
            ],
            out_specs=pl.BlockSpec((Q2, HID), lambda i: (i, 0)),
            out_shape=jax.ShapeDtypeStruct((N, HID), jnp.float32),
        )(coords_s16, v, cj, vj, w_r, w_i, w_j, b1,
          gk2_w, b2, w3r, b3, W_w, ln_scale[t][None, :], ln_bias[t][None, :])

    vjf = _gather_rows(v, idx_flat)
    scores_s, w_ij_s = pl.pallas_call(
        _final_body,
        grid=(N // Q2,),
        in_specs=row_specs + green_specs + [
            _wspec((HID, HID // 2)), _wspec((1, HID // 2)),
            _wspec((1, HID // 2)), _wspec((1, 1)),
        ],
        out_specs=[
            pl.BlockSpec((Q2, 1), lambda i: (i, 0)),
            pl.BlockSpec((Q2, K), lambda i: (i, 0)),
        ],
        out_shape=[
            jax.ShapeDtypeStruct((N, 1), jnp.float32),
            jax.ShapeDtypeStruct((N, K), jnp.float32),
        ],
    )(coords_s16, v, cj, vjf, w_r, w_i, w_j, b1, gk2_w, b2, w3r, b3,
      proj1_w, proj1_b[None, :], proj2_w.T, proj2_b[None, :])

    big = jnp.concatenate(
        [v, w_ij_s, idxvals_f,
         jnp.broadcast_to(scores_s, (N, 16))], axis=1)
    og = _gather_rows(big, inv)
    v = og[:, 0:HID]
    w_ij = og[:, HID:HID + K]
    idx = og[:, 48:64].astype(jnp.int32)
    scores = og[:, 64:65]
    return (scores, idx, w_ij, v)
